# baseline (speedup 1.0000x reference)
import functools

import jax
import jax.numpy as jnp
from jax import lax
from jax.experimental import pallas as pl
from jax.experimental.pallas import tpu as pltpu
from jax.experimental.pallas import tpu_sc as plsc

EMBED_DIM = 128
CHW = 64
DRING = 10
LAG = 6


@functools.cache
def _build(nw, nch, nc):
    mesh = plsc.VectorSubcoreMesh(core_axis_name="c", subcore_axis_name="s")
    rounds = nch // DRING
    assert rounds * DRING == nch

    @functools.partial(
        pl.kernel,
        out_type=jax.ShapeDtypeStruct((nw, nch, CHW, EMBED_DIM), jnp.float32),
        mesh=mesh,
        scratch_types=[
            pltpu.VMEM((nch, CHW), jnp.int32),
            pltpu.VMEM((DRING, CHW, EMBED_DIM), jnp.float32),
            pltpu.SemaphoreType.DMA((DRING,)),
            pltpu.SemaphoreType.DMA((DRING,)),
        ],
    )
    def gather_kernel(table_hbm, seq_hbm, out_hbm, idx_v, bufs, semg, semw):
        wid = lax.axis_index("s") * nc + lax.axis_index("c")
        pltpu.sync_copy(seq_hbm.at[wid], idx_v)

        def fire_g(t, b):
            pltpu.async_copy(table_hbm.at[idx_v.at[t]], bufs.at[b], semg.at[b])

        def drain_g(t, b):
            pltpu.make_async_copy(
                table_hbm.at[idx_v.at[t]], bufs.at[b], semg.at[b]).wait()

        def fire_w(u, su):
            pltpu.async_copy(bufs.at[su], out_hbm.at[wid, u], semw.at[su])

        def drain_w(u, su):
            pltpu.make_async_copy(
                bufs.at[su], out_hbm.at[wid, u], semw.at[su]).wait()

        for b in range(DRING):
            fire_g(b, b)
            if b >= LAG:
                u = b - LAG
                drain_g(u, u % DRING)
                fire_w(u, u % DRING)

        def round_body(r, carry):
            for b in range(DRING):
                t = r * DRING + b
                drain_w(t - DRING, b)
                fire_g(t, b)
                u = t - LAG
                su = (b - LAG) % DRING
                drain_g(u, su)
                fire_w(u, su)
            return carry

        lax.fori_loop(1, rounds, round_body, 0)

        for u in range(nch - LAG, nch):
            su = u % DRING
            drain_g(u, su)
            fire_w(u, su)
        for b in range(DRING):
            drain_w(nch - DRING + b, b)

    return gather_kernel


def kernel(sequence, table):
    batch, hist = sequence.shape
    total = batch * hist
    mesh = plsc.VectorSubcoreMesh(core_axis_name="c", subcore_axis_name="s")
    nw = mesh.num_cores * mesh.num_subcores
    nch = total // (nw * CHW)
    assert nch * nw * CHW == total
    seq = sequence.astype(jnp.int32).reshape(nw, nch, CHW)
    out = _build(nw, nch, mesh.num_cores)(table, seq)
    return out.reshape(batch, hist, EMBED_DIM)


# speedup vs baseline: 1.7840x; 1.7840x over previous
import functools

import jax
import jax.numpy as jnp
from jax import lax
from jax.experimental import pallas as pl
from jax.experimental.pallas import tpu as pltpu
from jax.experimental.pallas import tpu_sc as plsc

EMBED_DIM = 128
DRING = 8
LAG = 4


@functools.cache
def _build(batch, hist, nw, nc):
    mesh = plsc.VectorSubcoreMesh(core_axis_name="c", subcore_axis_name="s")
    bpw = batch // nw
    rounds = bpw // DRING
    assert rounds * DRING == bpw

    @functools.partial(
        pl.kernel,
        out_type=jax.ShapeDtypeStruct((batch, hist, EMBED_DIM), jnp.float32),
        mesh=mesh,
        scratch_types=[
            pltpu.VMEM((bpw, hist), jnp.int32),
            pltpu.VMEM((DRING, hist, EMBED_DIM), jnp.float32),
            pltpu.SemaphoreType.DMA((DRING,)),
            pltpu.SemaphoreType.DMA((DRING,)),
        ],
    )
    def gather_kernel(table_hbm, seq_hbm, out_hbm, idx_v, bufs, semg, semw):
        wid = lax.axis_index("s") * nc + lax.axis_index("c")
        base = wid * bpw
        pltpu.sync_copy(seq_hbm.at[pl.ds(base, bpw)], idx_v)

        def fire_g(t, b):
            pltpu.async_copy(table_hbm.at[idx_v.at[t]], bufs.at[b], semg.at[b])

        def drain_g(t, b):
            pltpu.make_async_copy(
                table_hbm.at[idx_v.at[t]], bufs.at[b], semg.at[b]).wait()

        def fire_w(u, su):
            pltpu.async_copy(bufs.at[su], out_hbm.at[base + u], semw.at[su])

        def drain_w(u, su):
            pltpu.make_async_copy(
                bufs.at[su], out_hbm.at[base + u], semw.at[su]).wait()

        for b in range(DRING):
            fire_g(b, b)
            if b >= LAG:
                u = b - LAG
                drain_g(u, u % DRING)
                fire_w(u, u % DRING)

        def round_body(r, carry):
            for b in range(DRING):
                t = r * DRING + b
                drain_w(t - DRING, b)
                fire_g(t, b)
                u = t - LAG
                su = (b - LAG) % DRING
                drain_g(u, su)
                fire_w(u, su)
            return carry

        lax.fori_loop(1, rounds, round_body, 0)

        for u in range(bpw - LAG, bpw):
            su = u % DRING
            drain_g(u, su)
            fire_w(u, su)
        for b in range(DRING):
            drain_w(bpw - DRING + b, b)

    return gather_kernel


def kernel(sequence, table):
    batch, hist = sequence.shape
    mesh = plsc.VectorSubcoreMesh(core_axis_name="c", subcore_axis_name="s")
    nw = mesh.num_cores * mesh.num_subcores
    assert batch % nw == 0
    seq = sequence.astype(jnp.int32)
    return _build(batch, hist, nw, mesh.num_cores)(table, seq)


# speedup vs baseline: 1.7885x; 1.0025x over previous
import functools

import jax
import jax.numpy as jnp
from jax import lax
from jax.experimental import pallas as pl
from jax.experimental.pallas import tpu as pltpu
from jax.experimental.pallas import tpu_sc as plsc

EMBED_DIM = 128
DRING = 8
LAG = 4


@functools.cache
def _build(batch, hist, nw, nc):
    mesh = plsc.VectorSubcoreMesh(core_axis_name="c", subcore_axis_name="s")
    bpw = batch // nw
    rounds = bpw // DRING
    assert rounds * DRING == bpw

    @functools.partial(
        pl.kernel,
        out_type=jax.ShapeDtypeStruct((batch, hist, EMBED_DIM), jnp.float32),
        mesh=mesh,
        compiler_params=pltpu.CompilerParams(use_tc_tiling_on_sc=True),
        scratch_types=[
            pltpu.VMEM((bpw, hist), jnp.int32),
            pltpu.VMEM((DRING, hist, EMBED_DIM), jnp.float32),
            pltpu.SemaphoreType.DMA((DRING,)),
            pltpu.SemaphoreType.DMA((DRING,)),
        ],
    )
    def gather_kernel(table_hbm, seq_hbm, out_hbm, idx_v, bufs, semg, semw):
        wid = lax.axis_index("s") * nc + lax.axis_index("c")
        base = wid * bpw
        pltpu.sync_copy(seq_hbm.at[pl.ds(base, bpw)], idx_v)

        def fire_g(t, b):
            pltpu.async_copy(table_hbm.at[idx_v.at[t]], bufs.at[b], semg.at[b])

        def drain_g(t, b):
            pltpu.make_async_copy(
                table_hbm.at[idx_v.at[t]], bufs.at[b], semg.at[b]).wait()

        def fire_w(u, su):
            pltpu.async_copy(bufs.at[su], out_hbm.at[base + u], semw.at[su])

        def drain_w(u, su):
            pltpu.make_async_copy(
                bufs.at[su], out_hbm.at[base + u], semw.at[su]).wait()

        for b in range(DRING):
            fire_g(b, b)
            if b >= LAG:
                u = b - LAG
                drain_g(u, u % DRING)
                fire_w(u, u % DRING)

        def round_body(r, carry):
            for b in range(DRING):
                t = r * DRING + b
                drain_w(t - DRING, b)
                fire_g(t, b)
                u = t - LAG
                su = (b - LAG) % DRING
                drain_g(u, su)
                fire_w(u, su)
            return carry

        lax.fori_loop(1, rounds, round_body, 0)

        for u in range(bpw - LAG, bpw):
            su = u % DRING
            drain_g(u, su)
            fire_w(u, su)
        for b in range(DRING):
            drain_w(bpw - DRING + b, b)

    return gather_kernel


def kernel(sequence, table):
    batch, hist = sequence.shape
    mesh = plsc.VectorSubcoreMesh(core_axis_name="c", subcore_axis_name="s")
    nw = mesh.num_cores * mesh.num_subcores
    assert batch % nw == 0
    seq = sequence.astype(jnp.int32)
    return _build(batch, hist, nw, mesh.num_cores)(table, seq)


# speedup vs baseline: 3.2213x; 1.8012x over previous
import functools

import jax
import jax.numpy as jnp
from jax import lax
from jax.experimental import pallas as pl
from jax.experimental.pallas import tpu as pltpu
from jax.experimental.pallas import tpu_sc as plsc

EMBED_DIM = 128
CHW = 128
DRING = 5
LAG = 3


@functools.cache
def _build(batch, hist, nw, nc):
    mesh = plsc.VectorSubcoreMesh(core_axis_name="c", subcore_axis_name="s")
    nch = hist
    rounds = nch // DRING
    assert rounds * DRING == nch and batch % (nw * CHW) == 0

    @functools.partial(
        pl.kernel,
        out_type=jax.ShapeDtypeStruct((hist, batch, EMBED_DIM), jnp.float32),
        mesh=mesh,
        scratch_types=[
            pltpu.VMEM((nch, CHW), jnp.int32),
            pltpu.VMEM((DRING, CHW, EMBED_DIM), jnp.float32),
            pltpu.SemaphoreType.DMA((DRING,)),
            pltpu.SemaphoreType.DMA((DRING,)),
        ],
    )
    def gather_kernel(table_hbm, seqt_hbm, out_hbm, idx_v, bufs, semg, semw):
        wid = lax.axis_index("s") * nc + lax.axis_index("c")
        base = wid * CHW
        pltpu.sync_copy(seqt_hbm.at[:, pl.ds(base, CHW)], idx_v)

        def fire_g(t, b):
            pltpu.async_copy(table_hbm.at[idx_v.at[t]], bufs.at[b], semg.at[b])

        def drain_g(t, b):
            pltpu.make_async_copy(
                table_hbm.at[idx_v.at[t]], bufs.at[b], semg.at[b]).wait()

        def fire_w(u, su):
            pltpu.async_copy(
                bufs.at[su], out_hbm.at[u, pl.ds(base, CHW)], semw.at[su])

        def drain_w(u, su):
            pltpu.make_async_copy(
                bufs.at[su], out_hbm.at[u, pl.ds(base, CHW)], semw.at[su]).wait()

        for b in range(DRING):
            fire_g(b, b)
            if b >= LAG:
                u = b - LAG
                drain_g(u, u % DRING)
                fire_w(u, u % DRING)

        def round_body(r, carry):
            for b in range(DRING):
                t = r * DRING + b
                drain_w(t - DRING, b)
                fire_g(t, b)
                u = t - LAG
                su = (b - LAG) % DRING
                drain_g(u, su)
                fire_w(u, su)
            return carry

        lax.fori_loop(1, rounds, round_body, 0)

        for u in range(nch - LAG, nch):
            su = u % DRING
            drain_g(u, su)
            fire_w(u, su)
        for b in range(DRING):
            drain_w(nch - DRING + b, b)

    return gather_kernel


def kernel(sequence, table):
    batch, hist = sequence.shape
    mesh = plsc.VectorSubcoreMesh(core_axis_name="c", subcore_axis_name="s")
    nw = mesh.num_cores * mesh.num_subcores
    seqt = sequence.astype(jnp.int32).T
    out = _build(batch, hist, nw, mesh.num_cores)(table, seqt)
    return out.transpose(1, 0, 2)


# speedup vs baseline: 3.2260x; 1.0015x over previous
import functools

import jax
import jax.numpy as jnp
from jax import lax
from jax.experimental import pallas as pl
from jax.experimental.pallas import tpu as pltpu
from jax.experimental.pallas import tpu_sc as plsc

EMBED_DIM = 128
CHW = 128
DRING = 5
LAG = 4


@functools.cache
def _build(batch, hist, nw, nc):
    mesh = plsc.VectorSubcoreMesh(core_axis_name="c", subcore_axis_name="s")
    nch = hist
    rounds = nch // DRING
    assert rounds * DRING == nch and batch % (nw * CHW) == 0

    @functools.partial(
        pl.kernel,
        out_type=jax.ShapeDtypeStruct((hist, batch, EMBED_DIM), jnp.float32),
        mesh=mesh,
        scratch_types=[
            pltpu.VMEM((nch, CHW), jnp.int32),
            pltpu.VMEM((DRING, CHW, EMBED_DIM), jnp.float32),
            pltpu.SemaphoreType.DMA((DRING,)),
            pltpu.SemaphoreType.DMA((DRING,)),
        ],
    )
    def gather_kernel(table_hbm, seqt_hbm, out_hbm, idx_v, bufs, semg, semw):
        wid = lax.axis_index("s") * nc + lax.axis_index("c")
        base = wid * CHW
        pltpu.sync_copy(seqt_hbm.at[:, pl.ds(base, CHW)], idx_v)

        def fire_g(t, b):
            pltpu.async_copy(table_hbm.at[idx_v.at[t]], bufs.at[b], semg.at[b])

        def drain_g(t, b):
            pltpu.make_async_copy(
                table_hbm.at[idx_v.at[t]], bufs.at[b], semg.at[b]).wait()

        def fire_w(u, su):
            pltpu.async_copy(
                bufs.at[su], out_hbm.at[u, pl.ds(base, CHW)], semw.at[su])

        def drain_w(u, su):
            pltpu.make_async_copy(
                bufs.at[su], out_hbm.at[u, pl.ds(base, CHW)], semw.at[su]).wait()

        for b in range(DRING):
            fire_g(b, b)
            if b >= LAG:
                u = b - LAG
                drain_g(u, u % DRING)
                fire_w(u, u % DRING)

        def round_body(r, carry):
            for b in range(DRING):
                t = r * DRING + b
                drain_w(t - DRING, b)
                fire_g(t, b)
                u = t - LAG
                su = (b - LAG) % DRING
                drain_g(u, su)
                fire_w(u, su)
            return carry

        lax.fori_loop(1, rounds, round_body, 0)

        for u in range(nch - LAG, nch):
            su = u % DRING
            drain_g(u, su)
            fire_w(u, su)
        for b in range(DRING):
            drain_w(nch - DRING + b, b)

    return gather_kernel


def kernel(sequence, table):
    batch, hist = sequence.shape
    mesh = plsc.VectorSubcoreMesh(core_axis_name="c", subcore_axis_name="s")
    nw = mesh.num_cores * mesh.num_subcores
    seqt = sequence.astype(jnp.int32).T
    out = _build(batch, hist, nw, mesh.num_cores)(table, seqt)
    return out.transpose(1, 0, 2)


# speedup vs baseline: 3.2261x; 1.0000x over previous
import functools

import jax
import jax.numpy as jnp
from jax import lax
from jax.experimental import pallas as pl
from jax.experimental.pallas import tpu as pltpu
from jax.experimental.pallas import tpu_sc as plsc

EMBED_DIM = 128
CHW = 128
DRING = 5
LAG = 3


@functools.cache
def _build(batch, hist, nw, nc):
    mesh = plsc.VectorSubcoreMesh(core_axis_name="c", subcore_axis_name="s")
    nch = hist
    rounds = nch // DRING
    assert rounds * DRING == nch and batch % (nw * CHW) == 0

    @functools.partial(
        pl.kernel,
        out_type=jax.ShapeDtypeStruct((hist, batch, EMBED_DIM), jnp.float32),
        mesh=mesh,
        scratch_types=[
            pltpu.VMEM((nch, CHW), jnp.int32),
            pltpu.VMEM((DRING, CHW, EMBED_DIM), jnp.float32),
            pltpu.SemaphoreType.DMA((DRING,)),
            pltpu.SemaphoreType.DMA((DRING,)),
        ],
    )
    def gather_kernel(table_hbm, seqt_hbm, out_hbm, idx_v, bufs, semg, semw):
        wid = lax.axis_index("s") * nc + lax.axis_index("c")
        base = wid * CHW
        pltpu.sync_copy(seqt_hbm.at[:, pl.ds(base, CHW)], idx_v)

        def fire_g(t, b):
            pltpu.async_copy(table_hbm.at[idx_v.at[t]], bufs.at[b], semg.at[b])

        def drain_g(t, b):
            pltpu.make_async_copy(
                table_hbm.at[idx_v.at[t]], bufs.at[b], semg.at[b]).wait()

        def fire_w(u, su):
            pltpu.async_copy(
                bufs.at[su], out_hbm.at[u, pl.ds(base, CHW)], semw.at[su])

        def drain_w(u, su):
            pltpu.make_async_copy(
                bufs.at[su], out_hbm.at[u, pl.ds(base, CHW)], semw.at[su]).wait()

        for b in range(DRING):
            fire_g(b, b)
            if b >= LAG:
                u = b - LAG
                drain_g(u, u % DRING)
                fire_w(u, u % DRING)

        def round_body(r, carry):
            for b in range(DRING):
                t = r * DRING + b
                drain_w(t - DRING, b)
                fire_g(t, b)
                u = t - LAG
                su = (b - LAG) % DRING
                drain_g(u, su)
                fire_w(u, su)
            return carry

        lax.fori_loop(1, rounds, round_body, 0)

        for u in range(nch - LAG, nch):
            su = u % DRING
            drain_g(u, su)
            fire_w(u, su)
        for b in range(DRING):
            drain_w(nch - DRING + b, b)

    return gather_kernel


def kernel(sequence, table):
    batch, hist = sequence.shape
    mesh = plsc.VectorSubcoreMesh(core_axis_name="c", subcore_axis_name="s")
    nw = mesh.num_cores * mesh.num_subcores
    seqt = sequence.astype(jnp.int32).T
    out = _build(batch, hist, nw, mesh.num_cores)(table, seqt)
    return out.transpose(1, 0, 2)
